# Initial kernel scaffold; baseline (speedup 1.0000x reference)
import jax
import jax.numpy as jnp
from jax.experimental import pallas as pl


def kernel(visual_features, W, b, emb):
    raise NotImplementedError("write your pallas kernel here")


# speedup vs baseline: 6.2931x; 6.2931x over previous
import dataclasses
import functools

import jax
import jax.numpy as jnp
from jax.experimental import pallas as pl
from jax.experimental.pallas import tpu as pltpu
from jax.experimental.pallas import tpu_sc as plsc

CAND = 16
TILE = 2048
NSL = TILE // 128
PERB = 3
NPL = 5
TOPK = 10
NEG_INF = float("-inf")
INT_MAX = 2**31 - 1


def _tile_topk_kernel(vf_ref, w_ref, vals_ref, idx_ref):
    j = pl.program_id(0)
    logits = jax.lax.dot_general(
        vf_ref[...], w_ref[...], (((1,), (1,)), ((), ())),
        preferred_element_type=jnp.float32,
        precision=jax.lax.Precision.DEFAULT)
    bm = logits.shape[0]
    lane = jax.lax.broadcasted_iota(jnp.int32, (bm, 128), 1)
    neg = jnp.full((bm, 128), NEG_INF, jnp.float32)
    zero = jnp.zeros((bm, 128), jnp.int32)
    m = [neg] * PERB
    c = [zero] * PERB
    for s in range(NSL):
        x = logits[:, s * 128:(s + 1) * 128]
        cx = lane + (s * 128)
        g = [x > m[t] for t in range(PERB)]
        for t in range(PERB - 1, 0, -1):
            m[t] = jnp.where(g[t - 1], m[t - 1], jnp.where(g[t], x, m[t]))
            c[t] = jnp.where(g[t - 1], c[t - 1], jnp.where(g[t], cx, c[t]))
        m[0] = jnp.where(g[0], x, m[0])
        c[0] = jnp.where(g[0], cx, c[0])
    base = j * TILE
    vals_ref[...] = jnp.concatenate(m, axis=1)
    idx_ref[...] = jnp.concatenate(c, axis=1) + base


def _merge_kernel(vals_ref, idx_ref, ov_ref, oi_ref, *, classes, width):
    bmb = ov_ref.shape[0]
    nsl = width // 128
    neg = jnp.full((bmb, 128), NEG_INF, jnp.float32)
    zero = jnp.zeros((bmb, 128), jnp.int32)
    m = [neg] * NPL
    c = [zero] * NPL
    for s in range(nsl):
        x = vals_ref[:, s * 128:(s + 1) * 128]
        cx = idx_ref[:, s * 128:(s + 1) * 128]
        x = jnp.where(cx < classes, x, NEG_INF)
        g = [x > m[t] for t in range(NPL)]
        for t in range(NPL - 1, 0, -1):
            m[t] = jnp.where(g[t - 1], m[t - 1], jnp.where(g[t], x, m[t]))
            c[t] = jnp.where(g[t - 1], c[t - 1], jnp.where(g[t], cx, c[t]))
        m[0] = jnp.where(g[0], x, m[0])
        c[0] = jnp.where(g[0], cx, c[0])
    ovs, ois = [], []
    for _ in range(CAND):
        top = m[0]
        for t in range(1, NPL):
            top = jnp.maximum(top, m[t])
        w = jnp.max(top, axis=1)
        wb = w[:, None]
        picks = [jnp.where(m[t] == wb, c[t], INT_MAX) for t in range(NPL)]
        pk = picks[0]
        for t in range(1, NPL):
            pk = jnp.minimum(pk, picks[t])
        pick = jnp.min(pk, axis=1)
        pb = pick[:, None]
        for t in range(NPL):
            m[t] = jnp.where((m[t] == wb) & (c[t] == pb), NEG_INF, m[t])
        ovs.append(w[:, None])
        ois.append(pick[:, None])
    ov_ref[...] = jnp.concatenate(ovs, axis=1)
    oi_ref[...] = jnp.concatenate(ois, axis=1)


def _sum_groups_kernel(g_ref, o_ref, *, k):
    rows, e = o_ref.shape
    g = g_ref[:, :e]
    o_ref[...] = jnp.sum(g.reshape(rows, k, e), axis=1)


def _sc_gather(idx2d, emb):
    n = idx2d.shape[1]
    e = emb.shape[1]
    mesh = plsc.VectorSubcoreMesh(core_axis_name="c", subcore_axis_name="s")
    cp = pltpu.CompilerParams()
    if "needs_layout_passes" in pltpu.CompilerParams.__dataclass_fields__:
        cp = dataclasses.replace(cp, needs_layout_passes=False)
    gw = 128

    @functools.partial(
        pl.kernel,
        out_type=jax.ShapeDtypeStruct((n, e), emb.dtype),
        mesh=mesh,
        compiler_params=cp)
    def k(i_hbm, e_hbm, o_hbm):
        def body(i_vmem, o_vmem):
            pltpu.sync_copy(e_hbm.at[i_vmem.at[0]], o_vmem)

        pltpu.emit_pipeline(
            body,
            grid=(n // gw,),
            in_specs=[pl.BlockSpec((1, gw), index_map=lambda i: (0, i))],
            out_specs=[pl.BlockSpec((gw, e), index_map=lambda i: (i, 0))],
            core_axis_name=("c", "s"),
            dimension_semantics=(pltpu.PARALLEL,),
        )(i_hbm, o_hbm)

    return k(idx2d, emb)


def kernel(visual_features, W, b, emb):
    batch, vis = visual_features.shape
    classes = W.shape[0]
    e = emb.shape[1]
    nt = -(-classes // TILE)
    cpad = nt * TILE
    w_padded = jnp.pad(W, ((0, cpad - classes), (0, 0)))

    bm = 512
    tpb = PERB * 128
    width = nt * tpb
    vals, idx = pl.pallas_call(
        _tile_topk_kernel,
        grid=(nt, batch // bm),
        in_specs=[
            pl.BlockSpec((bm, vis), lambda j, i: (i, 0)),
            pl.BlockSpec((TILE, vis), lambda j, i: (j, 0)),
        ],
        out_specs=[
            pl.BlockSpec((bm, tpb), lambda j, i: (i, j)),
            pl.BlockSpec((bm, tpb), lambda j, i: (i, j)),
        ],
        out_shape=[
            jax.ShapeDtypeStruct((batch, width), jnp.float32),
            jax.ShapeDtypeStruct((batch, width), jnp.int32),
        ],
    )(visual_features, w_padded)

    bmb = 32
    v16, i16 = pl.pallas_call(
        functools.partial(_merge_kernel, classes=classes, width=width),
        grid=(batch // bmb,),
        in_specs=[
            pl.BlockSpec((bmb, width), lambda i: (i, 0)),
            pl.BlockSpec((bmb, width), lambda i: (i, 0)),
        ],
        out_specs=[
            pl.BlockSpec((bmb, CAND), lambda i: (i, 0)),
            pl.BlockSpec((bmb, CAND), lambda i: (i, 0)),
        ],
        out_shape=[
            jax.ShapeDtypeStruct((batch, CAND), jnp.float32),
            jax.ShapeDtypeStruct((batch, CAND), jnp.int32),
        ],
    )(vals, idx)

    tags = jax.nn.sigmoid(v16)
    _, srt_idx = jax.lax.sort((jnp.negative(tags), i16), dimension=1,
                              num_keys=2)
    top_k_classes = srt_idx[:, :TOPK]

    emb_padded = jnp.pad(emb, ((0, 0), (0, 128 - e)))
    gathered = _sc_gather(top_k_classes.reshape(1, batch * TOPK), emb_padded)

    bms = 1024
    embeddings = pl.pallas_call(
        functools.partial(_sum_groups_kernel, k=TOPK),
        grid=(batch // bms,),
        in_specs=[pl.BlockSpec((bms * TOPK, 128), lambda i: (i, 0))],
        out_specs=pl.BlockSpec((bms, e), lambda i: (i, 0)),
        out_shape=jax.ShapeDtypeStruct((batch, e), jnp.float32),
    )(gathered)

    return (top_k_classes, embeddings)


# speedup vs baseline: 6.7562x; 1.0736x over previous
import dataclasses
import functools

import jax
import jax.numpy as jnp
from jax.experimental import pallas as pl
from jax.experimental.pallas import tpu as pltpu
from jax.experimental.pallas import tpu_sc as plsc

CAND = 16
TILE = 2048
NSL = TILE // 128
PERB = 3
NPL = 5
TOPK = 10
NEG_INF = float("-inf")
INT_MAX = 2**31 - 1


def _tile_topk_kernel(vf_ref, w_ref, vals_ref, idx_ref):
    j = pl.program_id(0)
    logits = jax.lax.dot_general(
        vf_ref[...], w_ref[...], (((1,), (1,)), ((), ())),
        preferred_element_type=jnp.float32,
        precision=jax.lax.Precision.DEFAULT)
    bm = logits.shape[0]
    lane = jax.lax.broadcasted_iota(jnp.int32, (bm, 128), 1)
    neg = jnp.full((bm, 128), NEG_INF, jnp.float32)
    zero = jnp.zeros((bm, 128), jnp.int32)
    m = [neg] * PERB
    c = [zero] * PERB
    for s in range(NSL):
        x = logits[:, s * 128:(s + 1) * 128]
        cx = lane + (s * 128)
        g = [x > m[t] for t in range(PERB)]
        for t in range(PERB - 1, 0, -1):
            m[t] = jnp.where(g[t - 1], m[t - 1], jnp.where(g[t], x, m[t]))
            c[t] = jnp.where(g[t - 1], c[t - 1], jnp.where(g[t], cx, c[t]))
        m[0] = jnp.where(g[0], x, m[0])
        c[0] = jnp.where(g[0], cx, c[0])
    base = j * TILE
    vals_ref[...] = jnp.concatenate(m, axis=1)
    idx_ref[...] = jnp.concatenate(c, axis=1) + base


def _merge_kernel(vals_ref, idx_ref, ov_ref, oi_ref, *, classes, width):
    bmb = ov_ref.shape[0]
    nsl = width // 128
    neg = jnp.full((bmb, 128), NEG_INF, jnp.float32)
    zero = jnp.zeros((bmb, 128), jnp.int32)
    m = [neg] * NPL
    c = [zero] * NPL
    for s in range(nsl):
        x = vals_ref[:, s * 128:(s + 1) * 128]
        cx = idx_ref[:, s * 128:(s + 1) * 128]
        x = jnp.where(cx < classes, x, NEG_INF)
        g = [x > m[t] for t in range(NPL)]
        for t in range(NPL - 1, 0, -1):
            m[t] = jnp.where(g[t - 1], m[t - 1], jnp.where(g[t], x, m[t]))
            c[t] = jnp.where(g[t - 1], c[t - 1], jnp.where(g[t], cx, c[t]))
        m[0] = jnp.where(g[0], x, m[0])
        c[0] = jnp.where(g[0], cx, c[0])
    ovs, ois = [], []
    for _ in range(CAND):
        top = m[0]
        for t in range(1, NPL):
            top = jnp.maximum(top, m[t])
        w = jnp.max(top, axis=1)
        wb = w[:, None]
        picks = [jnp.where(m[t] == wb, c[t], INT_MAX) for t in range(NPL)]
        pk = picks[0]
        for t in range(1, NPL):
            pk = jnp.minimum(pk, picks[t])
        pick = jnp.min(pk, axis=1)
        pb = pick[:, None]
        for t in range(NPL):
            m[t] = jnp.where((m[t] == wb) & (c[t] == pb), NEG_INF, m[t])
        ovs.append(w[:, None])
        ois.append(pick[:, None])
    ov_ref[...] = jnp.concatenate(ovs, axis=1)
    oi_ref[...] = jnp.concatenate(ois, axis=1)


def _sum_groups_kernel(g_ref, o_ref, *, k):
    rows, e = o_ref.shape
    g = g_ref[:, :e]
    o_ref[...] = jnp.sum(g.reshape(rows, k, e), axis=1)


def _sc_gather(idx2d, emb):
    n = idx2d.shape[1]
    e = emb.shape[1]
    mesh = plsc.VectorSubcoreMesh(core_axis_name="c", subcore_axis_name="s")
    cp = pltpu.CompilerParams()
    if "needs_layout_passes" in pltpu.CompilerParams.__dataclass_fields__:
        cp = dataclasses.replace(cp, needs_layout_passes=False)
    gw = 128

    @functools.partial(
        pl.kernel,
        out_type=jax.ShapeDtypeStruct((n, e), emb.dtype),
        mesh=mesh,
        compiler_params=cp)
    def k(i_hbm, e_hbm, o_hbm):
        def body(i_vmem, o_vmem):
            pltpu.sync_copy(e_hbm.at[i_vmem.at[0]], o_vmem)

        pltpu.emit_pipeline(
            body,
            grid=(n // gw,),
            in_specs=[pl.BlockSpec((1, gw), index_map=lambda i: (0, i))],
            out_specs=[pl.BlockSpec((gw, e), index_map=lambda i: (i, 0))],
            core_axis_name=("c", "s"),
            dimension_semantics=(pltpu.PARALLEL,),
        )(i_hbm, o_hbm)

    return k(idx2d, emb)


def kernel(visual_features, W, b, emb):
    batch, vis = visual_features.shape
    classes = W.shape[0]
    e = emb.shape[1]
    nt = -(-classes // TILE)
    cpad = nt * TILE
    w_padded = jnp.pad(W, ((0, cpad - classes), (0, 0)))

    bm = 1024
    tpb = PERB * 128
    width = nt * tpb
    vals, idx = pl.pallas_call(
        _tile_topk_kernel,
        grid=(nt, batch // bm),
        in_specs=[
            pl.BlockSpec((bm, vis), lambda j, i: (i, 0)),
            pl.BlockSpec((TILE, vis), lambda j, i: (j, 0)),
        ],
        out_specs=[
            pl.BlockSpec((bm, tpb), lambda j, i: (i, j)),
            pl.BlockSpec((bm, tpb), lambda j, i: (i, j)),
        ],
        out_shape=[
            jax.ShapeDtypeStruct((batch, width), jnp.float32),
            jax.ShapeDtypeStruct((batch, width), jnp.int32),
        ],
    )(visual_features, w_padded)

    bmb = 32
    v16, i16 = pl.pallas_call(
        functools.partial(_merge_kernel, classes=classes, width=width),
        grid=(batch // bmb,),
        in_specs=[
            pl.BlockSpec((bmb, width), lambda i: (i, 0)),
            pl.BlockSpec((bmb, width), lambda i: (i, 0)),
        ],
        out_specs=[
            pl.BlockSpec((bmb, CAND), lambda i: (i, 0)),
            pl.BlockSpec((bmb, CAND), lambda i: (i, 0)),
        ],
        out_shape=[
            jax.ShapeDtypeStruct((batch, CAND), jnp.float32),
            jax.ShapeDtypeStruct((batch, CAND), jnp.int32),
        ],
    )(vals, idx)

    tags = jax.nn.sigmoid(v16)
    _, srt_idx = jax.lax.sort((jnp.negative(tags), i16), dimension=1,
                              num_keys=2)
    top_k_classes = srt_idx[:, :TOPK]

    emb_padded = jnp.pad(emb, ((0, 0), (0, 128 - e)))
    gathered = _sc_gather(top_k_classes.reshape(1, batch * TOPK), emb_padded)

    bms = 1024
    embeddings = pl.pallas_call(
        functools.partial(_sum_groups_kernel, k=TOPK),
        grid=(batch // bms,),
        in_specs=[pl.BlockSpec((bms * TOPK, 128), lambda i: (i, 0))],
        out_specs=pl.BlockSpec((bms, e), lambda i: (i, 0)),
        out_shape=jax.ShapeDtypeStruct((batch, e), jnp.float32),
    )(gathered)

    return (top_k_classes, embeddings)


# speedup vs baseline: 7.8781x; 1.1661x over previous
import dataclasses
import functools

import jax
import jax.numpy as jnp
from jax.experimental import pallas as pl
from jax.experimental.pallas import tpu as pltpu
from jax.experimental.pallas import tpu_sc as plsc

CAND = 16
TILE = 2048
NSL = TILE // 128
PERB = 3
NPL = 5
TOPK = 10
NEG_INF = float("-inf")
INT_MAX = 2**31 - 1


def _tile_topk_kernel(vf_ref, w_ref, vals_ref, idx_ref):
    j = pl.program_id(0)
    logits = jax.lax.dot_general(
        vf_ref[...], w_ref[...], (((1,), (1,)), ((), ())),
        preferred_element_type=jnp.float32,
        precision=jax.lax.Precision.DEFAULT)
    bm = logits.shape[0]
    lane = jax.lax.broadcasted_iota(jnp.int32, (bm, 128), 1)
    neg = jnp.full((bm, 128), NEG_INF, jnp.float32)
    zero = jnp.zeros((bm, 128), jnp.int32)
    m = [neg] * PERB
    c = [zero] * PERB
    for s in range(NSL):
        x = logits[:, s * 128:(s + 1) * 128]
        cx = lane + (s * 128)
        g = [x > m[t] for t in range(PERB)]
        for t in range(PERB - 1, 0, -1):
            m[t] = jnp.where(g[t - 1], m[t - 1], jnp.where(g[t], x, m[t]))
            c[t] = jnp.where(g[t - 1], c[t - 1], jnp.where(g[t], cx, c[t]))
        m[0] = jnp.where(g[0], x, m[0])
        c[0] = jnp.where(g[0], cx, c[0])
    base = j * TILE
    vals_ref[...] = jnp.concatenate(m, axis=1)
    idx_ref[...] = jnp.concatenate(c, axis=1) + base


def _merge_kernel(vals_ref, idx_ref, ov_ref, oi_ref, *, classes, width):
    bmb = ov_ref.shape[0]
    nsl = width // 128
    neg = jnp.full((bmb, 128), NEG_INF, jnp.float32)
    zero = jnp.zeros((bmb, 128), jnp.int32)
    m = [neg] * NPL
    c = [zero] * NPL
    for s in range(nsl):
        x = vals_ref[:, s * 128:(s + 1) * 128]
        cx = idx_ref[:, s * 128:(s + 1) * 128]
        x = jnp.where(cx < classes, x, NEG_INF)
        g = [x > m[t] for t in range(NPL)]
        for t in range(NPL - 1, 0, -1):
            m[t] = jnp.where(g[t - 1], m[t - 1], jnp.where(g[t], x, m[t]))
            c[t] = jnp.where(g[t - 1], c[t - 1], jnp.where(g[t], cx, c[t]))
        m[0] = jnp.where(g[0], x, m[0])
        c[0] = jnp.where(g[0], cx, c[0])
    ovs, ois = [], []
    for _ in range(CAND):
        top = m[0]
        for t in range(1, NPL):
            top = jnp.maximum(top, m[t])
        w = jnp.max(top, axis=1)
        wb = w[:, None]
        picks = [jnp.where(m[t] == wb, c[t], INT_MAX) for t in range(NPL)]
        pk = picks[0]
        for t in range(1, NPL):
            pk = jnp.minimum(pk, picks[t])
        pick = jnp.min(pk, axis=1)
        pb = pick[:, None]
        for t in range(NPL):
            m[t] = jnp.where((m[t] == wb) & (c[t] == pb), NEG_INF, m[t])
        ovs.append(w[:, None])
        ois.append(pick[:, None])
    ov_ref[...] = jnp.concatenate(ovs, axis=1)
    oi_ref[...] = jnp.concatenate(ois, axis=1)


def _sum_groups_kernel(g_ref, o_ref, *, k):
    rows, e = o_ref.shape
    g = g_ref[:, :e]
    o_ref[...] = jnp.sum(g.reshape(rows, k, e), axis=1)


def _sc_gather(idx2d, emb):
    n = idx2d.shape[1]
    e = emb.shape[1]
    mesh = plsc.VectorSubcoreMesh(core_axis_name="c", subcore_axis_name="s")
    cp = pltpu.CompilerParams()
    if "needs_layout_passes" in pltpu.CompilerParams.__dataclass_fields__:
        cp = dataclasses.replace(cp, needs_layout_passes=False)
    gw = 128

    @functools.partial(
        pl.kernel,
        out_type=jax.ShapeDtypeStruct((n, e), emb.dtype),
        mesh=mesh,
        compiler_params=cp)
    def k(i_hbm, e_hbm, o_hbm):
        def body(i_vmem, o_vmem):
            pltpu.sync_copy(e_hbm.at[i_vmem.at[0]], o_vmem)

        pltpu.emit_pipeline(
            body,
            grid=(n // gw,),
            in_specs=[pl.BlockSpec((1, gw), index_map=lambda i: (0, i))],
            out_specs=[pl.BlockSpec((gw, e), index_map=lambda i: (i, 0))],
            core_axis_name=("c", "s"),
            dimension_semantics=(pltpu.PARALLEL,),
        )(i_hbm, o_hbm)

    return k(idx2d, emb)


def kernel(visual_features, W, b, emb):
    batch, vis = visual_features.shape
    classes = W.shape[0]
    e = emb.shape[1]
    nt = -(-classes // TILE)
    cpad = nt * TILE
    w_padded = jnp.pad(W, ((0, cpad - classes), (0, 0)))

    bm = 1024
    tpb = PERB * 128
    width = nt * tpb
    vals, idx = pl.pallas_call(
        _tile_topk_kernel,
        grid=(nt, batch // bm),
        in_specs=[
            pl.BlockSpec((bm, vis), lambda j, i: (i, 0)),
            pl.BlockSpec((TILE, vis), lambda j, i: (j, 0)),
        ],
        out_specs=[
            pl.BlockSpec((bm, tpb), lambda j, i: (i, j)),
            pl.BlockSpec((bm, tpb), lambda j, i: (i, j)),
        ],
        out_shape=[
            jax.ShapeDtypeStruct((batch, width), jnp.float32),
            jax.ShapeDtypeStruct((batch, width), jnp.int32),
        ],
    )(visual_features, w_padded)

    bmb = 64
    v16, i16 = pl.pallas_call(
        functools.partial(_merge_kernel, classes=classes, width=width),
        grid=(batch // bmb,),
        in_specs=[
            pl.BlockSpec((bmb, width), lambda i: (i, 0)),
            pl.BlockSpec((bmb, width), lambda i: (i, 0)),
        ],
        out_specs=[
            pl.BlockSpec((bmb, CAND), lambda i: (i, 0)),
            pl.BlockSpec((bmb, CAND), lambda i: (i, 0)),
        ],
        out_shape=[
            jax.ShapeDtypeStruct((batch, CAND), jnp.float32),
            jax.ShapeDtypeStruct((batch, CAND), jnp.int32),
        ],
    )(vals, idx)

    tags = jax.nn.sigmoid(v16)
    _, srt_idx = jax.lax.sort((jnp.negative(tags), i16), dimension=1,
                              num_keys=2)
    top_k_classes = srt_idx[:, :TOPK]

    emb_padded = jnp.pad(emb, ((0, 0), (0, 128 - e)))
    gathered = _sc_gather(top_k_classes.reshape(1, batch * TOPK), emb_padded)

    bms = 1024
    embeddings = pl.pallas_call(
        functools.partial(_sum_groups_kernel, k=TOPK),
        grid=(batch // bms,),
        in_specs=[pl.BlockSpec((bms * TOPK, 128), lambda i: (i, 0))],
        out_specs=pl.BlockSpec((bms, e), lambda i: (i, 0)),
        out_shape=jax.ShapeDtypeStruct((batch, e), jnp.float32),
    )(gathered)

    return (top_k_classes, embeddings)
